# Initial kernel scaffold; baseline (speedup 1.0000x reference)
import jax
import jax.numpy as jnp
from jax.experimental import pallas as pl


def kernel(x, edge_index, edge_weight, W, b):
    raise NotImplementedError("write your pallas kernel here")


# speedup vs baseline: 3.8027x; 3.8027x over previous
import functools

import jax
import jax.numpy as jnp
from jax import lax
from jax.experimental import pallas as pl
from jax.experimental.pallas import tpu as pltpu
from jax.experimental.pallas import tpu_sc as plsc

NC = 2
NS = 16
NW = NC * NS
EB = 128
LG = 8


def _sc_aggregate(n_pad, d, bpt):
    rows_per_tile = n_pad // NS
    cchunk = EB
    nchunk = rows_per_tile // cchunk

    def body(x_hbm, src_hbm, dst_hbm, w_hbm, out_hbm,
             src0, dst0, w0, rows0, src1, dst1, w1, rows1,
             accum, semE0, semE1, semG0, semG1):
        c = lax.axis_index("c")
        s = lax.axis_index("s")
        wid = c * NS + s

        sets = ((src0, dst0, w0, rows0, semE0, semG0),
                (src1, dst1, w1, rows1, semE1, semG1))

        zero = jnp.zeros((16,), jnp.float32)

        def zrow(r, carry):
            for j in range(LG):
                rows0[r, pl.ds(j * 16, 16)] = zero
            return carry

        lax.fori_loop(0, EB, zrow, 0)
        for k in range(nchunk):
            r0 = s * rows_per_tile + k * cchunk
            pltpu.sync_copy(rows0.at[pl.ds(0, cchunk), :],
                            accum.at[pl.ds(r0, cchunk), :])
        plsc.subcore_barrier()

        def fetch_edges(bg, st):
            e0 = bg * EB
            pltpu.make_async_copy(src_hbm.at[pl.ds(e0, EB)], st[0], st[4]).start()
            pltpu.make_async_copy(dst_hbm.at[pl.ds(e0, EB)], st[1], st[4]).start()
            pltpu.make_async_copy(w_hbm.at[pl.ds(e0, EB)], st[2], st[4]).start()

        def wait_edges(st):
            pltpu.make_async_copy(src_hbm.at[pl.ds(0, EB)], st[0], st[4]).wait()
            pltpu.make_async_copy(dst_hbm.at[pl.ds(0, EB)], st[1], st[4]).wait()
            pltpu.make_async_copy(w_hbm.at[pl.ds(0, EB)], st[2], st[4]).wait()

        def start_gather(st):
            pltpu.make_async_copy(x_hbm.at[st[0]], st[3], st[5]).start()

        def wait_gather(st):
            pltpu.make_async_copy(x_hbm.at[st[0]], st[3], st[5]).wait()

        def scale(st):
            w_r, rows_r = st[2], st[3]

            def gbody(g, carry):
                wv = w_r[pl.ds(pl.multiple_of(g * 16, 16), 16)]
                for el in range(16):
                    e = g * 16 + el
                    wb = lax.gather(
                        wv, jnp.full((16, 1), el, jnp.int32),
                        lax.GatherDimensionNumbers(
                            offset_dims=(), collapsed_slice_dims=(0,),
                            start_index_map=(0,)),
                        slice_sizes=(1,),
                        mode=lax.GatherScatterMode.PROMISE_IN_BOUNDS)
                    for j in range(LG):
                        sl = pl.ds(j * 16, 16)
                        rows_r[e, sl] = rows_r[e, sl] * wb
                return carry

            lax.fori_loop(0, EB // 16, gbody, 0)

        def scatter_add(st):
            pltpu.sync_copy(st[3], accum.at[st[1]], add=True)

        g0 = wid * bpt
        fetch_edges(g0, sets[0])

        def loop_body(i, carry):
            b0 = g0 + 2 * i
            wait_edges(sets[0])
            start_gather(sets[0])
            fetch_edges(b0 + 1, sets[1])
            wait_gather(sets[0])
            scale(sets[0])
            scatter_add(sets[0])
            wait_edges(sets[1])
            start_gather(sets[1])

            @pl.when(i < bpt // 2 - 1)
            def _():
                fetch_edges(b0 + 2, sets[0])

            wait_gather(sets[1])
            scale(sets[1])
            scatter_add(sets[1])
            return carry

        lax.fori_loop(0, bpt // 2, loop_body, 0)
        plsc.subcore_barrier()

        for k in range(nchunk):
            r0 = s * rows_per_tile + k * cchunk
            pltpu.sync_copy(accum.at[pl.ds(r0, cchunk), :],
                            out_hbm.at[c, pl.ds(r0, cchunk), :])

    mesh = plsc.VectorSubcoreMesh(core_axis_name="c", subcore_axis_name="s",
                                  num_cores=NC, num_subcores=NS)
    return pl.kernel(
        body,
        out_type=jax.ShapeDtypeStruct((NC, n_pad, d), jnp.float32),
        mesh=mesh,
        scratch_types=[
            pltpu.VMEM((EB,), jnp.int32),
            pltpu.VMEM((EB,), jnp.int32),
            pltpu.VMEM((EB,), jnp.float32),
            pltpu.VMEM((EB, d), jnp.float32),
            pltpu.VMEM((EB,), jnp.int32),
            pltpu.VMEM((EB,), jnp.int32),
            pltpu.VMEM((EB,), jnp.float32),
            pltpu.VMEM((EB, d), jnp.float32),
            pltpu.VMEM_SHARED((n_pad, d), jnp.float32),
            pltpu.SemaphoreType.DMA,
            pltpu.SemaphoreType.DMA,
            pltpu.SemaphoreType.DMA,
            pltpu.SemaphoreType.DMA,
        ],
    )


def _tc_body(p_ref, w_ref, b_ref, o_ref):
    acc = p_ref[0] + p_ref[1]
    o_ref[...] = (
        jnp.dot(acc, w_ref[...], preferred_element_type=jnp.float32)
        + b_ref[...]
    )


@jax.jit
def kernel(x, edge_index, edge_weight, W, b):
    n, d_in = x.shape
    d_out = W.shape[1]
    e = edge_weight.shape[0]

    src = edge_index[0].astype(jnp.int32)
    dst = edge_index[1].astype(jnp.int32)
    w = edge_weight.astype(jnp.float32)

    ept = EB * NW
    bpt = 2 * -(-e // (ept * 2))
    e_pad = bpt * ept
    src = jnp.pad(src, (0, e_pad - e))
    dst = jnp.pad(dst, (0, e_pad - e))
    w = jnp.pad(w, (0, e_pad - e))

    n_pad = -(-n // (NS * EB)) * NS * EB
    partials = _sc_aggregate(n_pad, d_in, bpt)(x, src, dst, w)

    rows_blk = 1000 if n % 1000 == 0 else n
    grid = n // rows_blk
    out = pl.pallas_call(
        _tc_body,
        grid=(grid,),
        in_specs=[
            pl.BlockSpec((NC, rows_blk, d_in), lambda i: (0, i, 0)),
            pl.BlockSpec((d_in, d_out), lambda i: (0, 0)),
            pl.BlockSpec((1, d_out), lambda i: (0, 0)),
        ],
        out_specs=pl.BlockSpec((rows_blk, d_out), lambda i: (i, 0)),
        out_shape=jax.ShapeDtypeStruct((n, d_out), jnp.float32),
    )(partials, W, b.reshape(1, d_out))
    return out


# speedup vs baseline: 7.6942x; 2.0234x over previous
import functools

import jax
import jax.numpy as jnp
from jax import lax
from jax.experimental import pallas as pl
from jax.experimental.pallas import tpu as pltpu
from jax.experimental.pallas import tpu_sc as plsc

NC = 2
NS = 16
NW = NC * NS
EB = 128
LG = 8


def _sc_aggregate(n_pad, d, bpt):
    rows_per_tile = n_pad // NS
    cchunk = EB
    nchunk = rows_per_tile // cchunk

    def body(x_hbm, src_hbm, dst_hbm, w_hbm, out_hbm,
             src0, dst0, w0, rows0, src1, dst1, w1, rows1,
             accum, semE0, semE1, semG0, semG1):
        c = lax.axis_index("c")
        s = lax.axis_index("s")
        wid = c * NS + s

        sets = ((src0, dst0, w0, rows0, semE0, semG0),
                (src1, dst1, w1, rows1, semE1, semG1))

        zero = jnp.zeros((16,), jnp.float32)

        def zrow(r, carry):
            for j in range(LG):
                rows0[r, pl.ds(j * 16, 16)] = zero
            return carry

        lax.fori_loop(0, EB, zrow, 0)
        for k in range(nchunk):
            r0 = s * rows_per_tile + k * cchunk
            pltpu.sync_copy(rows0.at[pl.ds(0, cchunk), :],
                            accum.at[pl.ds(r0, cchunk), :])
        plsc.subcore_barrier()

        def fetch_edges(bg, st):
            e0 = bg * EB
            pltpu.make_async_copy(src_hbm.at[pl.ds(e0, EB)], st[0], st[4]).start()
            pltpu.make_async_copy(dst_hbm.at[pl.ds(e0, EB)], st[1], st[4]).start()
            pltpu.make_async_copy(w_hbm.at[pl.ds(e0, EB)], st[2], st[4]).start()

        def wait_edges(st):
            pltpu.make_async_copy(src_hbm.at[pl.ds(0, EB)], st[0], st[4]).wait()
            pltpu.make_async_copy(dst_hbm.at[pl.ds(0, EB)], st[1], st[4]).wait()
            pltpu.make_async_copy(w_hbm.at[pl.ds(0, EB)], st[2], st[4]).wait()

        def start_gather(st):
            pltpu.make_async_copy(x_hbm.at[st[0]], st[3], st[5]).start()

        def wait_gather(st):
            pltpu.make_async_copy(x_hbm.at[st[0]], st[3], st[5]).wait()

        def scale(st):
            w_r, rows_r = st[2], st[3]

            def gbody(g, carry):
                wv = w_r[pl.ds(pl.multiple_of(g * 16, 16), 16)]
                for el in range(16):
                    e = g * 16 + el
                    wb = lax.gather(
                        wv, jnp.full((16, 1), el, jnp.int32),
                        lax.GatherDimensionNumbers(
                            offset_dims=(), collapsed_slice_dims=(0,),
                            start_index_map=(0,)),
                        slice_sizes=(1,),
                        mode=lax.GatherScatterMode.PROMISE_IN_BOUNDS)
                    for j in range(LG):
                        sl = pl.ds(j * 16, 16)
                        rows_r[e, sl] = rows_r[e, sl] * wb
                return carry

            lax.fori_loop(0, EB // 16, gbody, 0)

        def scatter_add(st):
            pltpu.sync_copy(st[3], accum.at[st[1]], add=True)

        g0 = wid * bpt
        fetch_edges(g0, sets[0])

        def loop_body(i, carry):
            b0 = g0 + 2 * i
            wait_edges(sets[0])
            start_gather(sets[0])
            fetch_edges(b0 + 1, sets[1])
            wait_gather(sets[0])
            scale(sets[0])
            scatter_add(sets[0])
            wait_edges(sets[1])
            start_gather(sets[1])

            @pl.when(i < bpt // 2 - 1)
            def _():
                fetch_edges(b0 + 2, sets[0])

            wait_gather(sets[1])
            scale(sets[1])
            scatter_add(sets[1])
            return carry

        lax.fori_loop(0, bpt // 2, loop_body, 0)
        plsc.subcore_barrier()

        for k in range(nchunk):
            r0 = s * rows_per_tile + k * cchunk
            pltpu.sync_copy(accum.at[pl.ds(r0, cchunk), :],
                            out_hbm.at[c, pl.ds(r0, cchunk), :])

    mesh = plsc.VectorSubcoreMesh(core_axis_name="c", subcore_axis_name="s",
                                  num_cores=NC, num_subcores=NS)
    return pl.kernel(
        body,
        out_type=jax.ShapeDtypeStruct((NC, n_pad, d), jnp.float32),
        mesh=mesh,
        scratch_types=[
            pltpu.VMEM((EB,), jnp.int32),
            pltpu.VMEM((EB,), jnp.int32),
            pltpu.VMEM((EB,), jnp.float32),
            pltpu.VMEM((EB, d), jnp.float32),
            pltpu.VMEM((EB,), jnp.int32),
            pltpu.VMEM((EB,), jnp.int32),
            pltpu.VMEM((EB,), jnp.float32),
            pltpu.VMEM((EB, d), jnp.float32),
            pltpu.VMEM_SHARED((n_pad, d), jnp.float32),
            pltpu.SemaphoreType.DMA,
            pltpu.SemaphoreType.DMA,
            pltpu.SemaphoreType.DMA,
            pltpu.SemaphoreType.DMA,
        ],
    )


def _tc_body(p_ref, w_ref, b_ref, o_ref):
    acc = p_ref[0] + p_ref[1]
    o_ref[...] = (
        jnp.dot(acc, w_ref[...], preferred_element_type=jnp.float32)
        + b_ref[...]
    )


@jax.jit
def kernel(x, edge_index, edge_weight, W, b):
    n, d_in = x.shape
    d_out = W.shape[1]
    e = edge_weight.shape[0]

    src = edge_index[0].astype(jnp.int32)
    dst = edge_index[1].astype(jnp.int32)
    w = edge_weight.astype(jnp.float32)

    ept = EB * NW
    bpt = 2 * -(-e // (ept * 2))
    e_pad = bpt * ept
    n_pad = -(-n // (NS * EB)) * NS * EB
    npe = e_pad - e
    fill = jnp.arange(npe, dtype=jnp.int32)
    src = jnp.concatenate([src, fill % n])
    dst = jnp.concatenate([dst, n + fill % (n_pad - n)])
    w = jnp.concatenate([w, jnp.zeros((npe,), jnp.float32)])

    partials = _sc_aggregate(n_pad, d_in, bpt)(x, src, dst, w)

    rows_blk = 1000 if n % 1000 == 0 else n
    grid = n // rows_blk
    out = pl.pallas_call(
        _tc_body,
        grid=(grid,),
        in_specs=[
            pl.BlockSpec((NC, rows_blk, d_in), lambda i: (0, i, 0)),
            pl.BlockSpec((d_in, d_out), lambda i: (0, 0)),
            pl.BlockSpec((1, d_out), lambda i: (0, 0)),
        ],
        out_specs=pl.BlockSpec((rows_blk, d_out), lambda i: (i, 0)),
        out_shape=jax.ShapeDtypeStruct((n, d_out), jnp.float32),
    )(partials, W, b.reshape(1, d_out))
    return out


# speedup vs baseline: 11.7191x; 1.5231x over previous
import functools

import jax
import jax.numpy as jnp
from jax import lax
from jax.experimental import pallas as pl
from jax.experimental.pallas import tpu as pltpu
from jax.experimental.pallas import tpu_sc as plsc

NC = 2
NS = 16
NW = NC * NS
EB = 128
LG = 8


def _sc_aggregate(n_pad, d, bpt):
    rows_per_tile = n_pad // NS
    cchunk = EB
    nchunk = rows_per_tile // cchunk

    def body(x_hbm, src_hbm, dst_hbm, w_hbm, out_hbm,
             src_all, dst_all, w_all, rows0, rows1,
             accum, semE, semG0, semG1):
        c = lax.axis_index("c")
        s = lax.axis_index("s")
        wid = c * NS + s

        hb = bpt // 2
        b0 = wid * bpt

        def fetch_half(h):
            o = b0 + h * hb
            pltpu.make_async_copy(src_hbm.at[pl.ds(o, hb), :], src_all, semE).start()
            pltpu.make_async_copy(dst_hbm.at[pl.ds(o, hb), :], dst_all, semE).start()
            pltpu.make_async_copy(w_hbm.at[pl.ds(o, hb), :], w_all, semE).start()

        def wait_half():
            pltpu.make_async_copy(src_hbm.at[pl.ds(b0, hb), :], src_all, semE).wait()
            pltpu.make_async_copy(dst_hbm.at[pl.ds(b0, hb), :], dst_all, semE).wait()
            pltpu.make_async_copy(w_hbm.at[pl.ds(b0, hb), :], w_all, semE).wait()

        fetch_half(0)

        zero = jnp.zeros((16,), jnp.float32)

        def zrow(r, carry):
            for j in range(LG):
                rows0[r, pl.ds(j * 16, 16)] = zero
            return carry

        lax.fori_loop(0, EB, zrow, 0)
        for k in range(nchunk):
            r0 = s * rows_per_tile + k * cchunk
            pltpu.sync_copy(rows0.at[pl.ds(0, cchunk), :],
                            accum.at[pl.ds(r0, cchunk), :])
        plsc.subcore_barrier()

        def start_gather(j, rows_r, sem):
            pltpu.make_async_copy(x_hbm.at[src_all.at[j]], rows_r, sem).start()

        def wait_gather(j, rows_r, sem):
            pltpu.make_async_copy(x_hbm.at[src_all.at[j]], rows_r, sem).wait()

        def scale(j, rows_r):
            def gbody(g, carry):
                wv = w_all[j, pl.ds(pl.multiple_of(g * 16, 16), 16)]
                for el in range(16):
                    e = g * 16 + el
                    wb = lax.gather(
                        wv, jnp.full((16, 1), el, jnp.int32),
                        lax.GatherDimensionNumbers(
                            offset_dims=(), collapsed_slice_dims=(0,),
                            start_index_map=(0,)),
                        slice_sizes=(1,),
                        mode=lax.GatherScatterMode.PROMISE_IN_BOUNDS)
                    for jj in range(LG):
                        sl = pl.ds(jj * 16, 16)
                        rows_r[e, sl] = rows_r[e, sl] * wb
                return carry

            lax.fori_loop(0, EB // 16, gbody, 0)

        def scatter_add(j, rows_r):
            pltpu.sync_copy(rows_r, accum.at[dst_all.at[j]], add=True)

        pairs = hb // 2

        def loop_body(i, carry):
            j = 2 * i
            start_gather(j + 1, rows1, semG1)
            wait_gather(j, rows0, semG0)
            scale(j, rows0)
            scatter_add(j, rows0)

            @pl.when(i < pairs - 1)
            def _():
                start_gather(j + 2, rows0, semG0)

            wait_gather(j + 1, rows1, semG1)
            scale(j + 1, rows1)
            scatter_add(j + 1, rows1)
            return carry

        for h in range(2):
            if h == 1:
                fetch_half(1)
            wait_half()
            start_gather(0, rows0, semG0)
            lax.fori_loop(0, pairs, loop_body, 0)
        plsc.subcore_barrier()

        for k in range(nchunk):
            r0 = s * rows_per_tile + k * cchunk
            pltpu.sync_copy(accum.at[pl.ds(r0, cchunk), :],
                            out_hbm.at[c, pl.ds(r0, cchunk), :])

    mesh = plsc.VectorSubcoreMesh(core_axis_name="c", subcore_axis_name="s",
                                  num_cores=NC, num_subcores=NS)
    return pl.kernel(
        body,
        out_type=jax.ShapeDtypeStruct((NC, n_pad, d), jnp.float32),
        mesh=mesh,
        scratch_types=[
            pltpu.VMEM((bpt // 2, EB), jnp.int32),
            pltpu.VMEM((bpt // 2, EB), jnp.int32),
            pltpu.VMEM((bpt // 2, EB), jnp.float32),
            pltpu.VMEM((EB, d), jnp.float32),
            pltpu.VMEM((EB, d), jnp.float32),
            pltpu.VMEM_SHARED((n_pad, d), jnp.float32),
            pltpu.SemaphoreType.DMA,
            pltpu.SemaphoreType.DMA,
            pltpu.SemaphoreType.DMA,
        ],
    )


def _tc_body(p_ref, w_ref, b_ref, o_ref):
    acc = p_ref[0] + p_ref[1]
    o_ref[...] = (
        jnp.dot(acc, w_ref[...], preferred_element_type=jnp.float32)
        + b_ref[...]
    )


@jax.jit
def kernel(x, edge_index, edge_weight, W, b):
    n, d_in = x.shape
    d_out = W.shape[1]
    e = edge_weight.shape[0]

    src = edge_index[0].astype(jnp.int32)
    dst = edge_index[1].astype(jnp.int32)
    w = edge_weight.astype(jnp.float32)

    ept = EB * NW
    bpt = 2 * -(-e // (ept * 2))
    e_pad = bpt * ept
    n_pad = -(-n // (NS * EB)) * NS * EB
    npe = e_pad - e
    fill = jnp.arange(npe, dtype=jnp.int32)
    src = jnp.concatenate([src, fill % n]).reshape(-1, EB)
    dst = jnp.concatenate([dst, n + fill % (n_pad - n)]).reshape(-1, EB)
    w = jnp.concatenate([w, jnp.zeros((npe,), jnp.float32)]).reshape(-1, EB)

    partials = _sc_aggregate(n_pad, d_in, bpt)(x, src, dst, w)

    rows_blk = 1000 if n % 1000 == 0 else n
    grid = n // rows_blk
    out = pl.pallas_call(
        _tc_body,
        grid=(grid,),
        in_specs=[
            pl.BlockSpec((NC, rows_blk, d_in), lambda i: (0, i, 0)),
            pl.BlockSpec((d_in, d_out), lambda i: (0, 0)),
            pl.BlockSpec((1, d_out), lambda i: (0, 0)),
        ],
        out_specs=pl.BlockSpec((rows_blk, d_out), lambda i: (i, 0)),
        out_shape=jax.ShapeDtypeStruct((n, d_out), jnp.float32),
    )(partials, W, b.reshape(1, d_out))
    return out
